# baseline (speedup 1.0000x reference)
import jax
import jax.numpy as jnp
import numpy as np
from jax.experimental import pallas as pl

_DIN = 48
_DOUT = 192
_DT = 16
_NPLANES = _DT // 4 + 2


def _interp_matrix(n_in: int, n_out: int) -> np.ndarray:
    m = np.zeros((n_in, n_out), dtype=np.float32)
    scale = n_in / n_out
    for o in range(n_out):
        c = (o + 0.5) * scale - 0.5
        i0 = int(np.floor(c))
        t = c - i0
        m[min(max(i0, 0), n_in - 1), o] += 1.0 - t
        m[min(max(i0 + 1, 0), n_in - 1), o] += t
    return m


def _body(a_ref, mht_ref, mwc_ref, o_ref):
    i = pl.program_id(0)
    mht = mht_ref[...]
    d0 = i * (_DT // 4) - 1
    u = [[] for _ in range(3)]
    for j in range(_NPLANES):
        dj = jnp.clip(d0 + j, 0, _DIN - 1)
        s2 = jnp.dot(mht, a_ref[dj], preferred_element_type=jnp.float32)
        for c in range(3):
            u[c].append(
                jnp.dot(s2, mwc_ref[c], preferred_element_type=jnp.float32)
            )
    for k in range(_DT):
        i0rel = (2 * k - 3) // 8
        frac = k * 0.25 - 0.375 - i0rel
        j0 = i0rel + 1
        for c in range(3):
            o_ref[c, k] = (1.0 - frac) * u[c][j0] + frac * u[c][j0 + 1]


@jax.jit
def _upsample(disp):
    a = jnp.reshape(disp, (_DIN, _DIN, 3 * _DIN))
    mw = _interp_matrix(_DIN, _DOUT)
    mht = jnp.asarray(mw.T)
    mwc = np.zeros((3, 3 * _DIN, _DOUT), dtype=np.float32)
    for c in range(3):
        mwc[c, c::3, :] = mw
    mwc = jnp.asarray(mwc)
    out = pl.pallas_call(
        _body,
        grid=(_DOUT // _DT,),
        in_specs=[
            pl.BlockSpec((_DIN, _DIN, 3 * _DIN), lambda i: (0, 0, 0)),
            pl.BlockSpec((_DOUT, _DIN), lambda i: (0, 0)),
            pl.BlockSpec((3, 3 * _DIN, _DOUT), lambda i: (0, 0, 0)),
        ],
        out_specs=pl.BlockSpec((3, _DT, _DOUT, _DOUT), lambda i: (0, i, 0, 0)),
        out_shape=jax.ShapeDtypeStruct((3, _DOUT, _DOUT, _DOUT), jnp.float32),
    )(a, mht, mwc)
    return jnp.reshape(out, (1, 3, _DOUT, _DOUT, _DOUT))


def kernel(kpts, disp, features_fixed):
    del kpts, features_fixed
    return _upsample(disp)
